# Initial kernel scaffold; baseline (speedup 1.0000x reference)
import jax
import jax.numpy as jnp
from jax.experimental import pallas as pl


def kernel(token_ids, embedding_matrix):
    raise NotImplementedError("write your pallas kernel here")


# speedup vs baseline: 1.8793x; 1.8793x over previous
import functools

import jax
import jax.numpy as jnp
from jax import lax
from jax.experimental import pallas as pl
from jax.experimental.pallas import tpu as pltpu
from jax.experimental.pallas import tpu_sc as plsc

EMB_DIM = 64
NUM_CORES = 2
NUM_SUBCORES = 16
NUM_WORKERS = NUM_CORES * NUM_SUBCORES
CHUNK = 128
NBUF = 4


@functools.partial(jax.jit, static_argnames=("n_total",))
def _emb_lookup(idx_flat, table, *, n_total):
    per_w = n_total // NUM_WORKERS
    n_chunks = per_w // CHUNK
    mesh = plsc.VectorSubcoreMesh(
        core_axis_name="c", subcore_axis_name="s",
        num_cores=NUM_CORES, num_subcores=NUM_SUBCORES)

    @functools.partial(
        pl.kernel,
        mesh=mesh,
        out_type=jax.ShapeDtypeStruct((n_total, EMB_DIM), jnp.float32),
        scratch_types=[
            pltpu.VMEM((per_w,), jnp.int32),
            pltpu.VMEM((NBUF, CHUNK, EMB_DIM), jnp.float32),
            pltpu.SemaphoreType.DMA((NBUF,)),
            pltpu.SemaphoreType.DMA((NBUF,)),
        ],
        compiler_params=pltpu.CompilerParams(use_tc_tiling_on_sc=False),
    )
    def emb_kernel(idx_hbm, table_hbm, out_hbm, idx_v, rows_v, g_sem, s_sem):
        wid = lax.axis_index("s") * NUM_CORES + lax.axis_index("c")
        base = wid * per_w
        pltpu.sync_copy(idx_hbm.at[pl.ds(base, per_w)], idx_v)

        def chunk_idx(j):
            off = pl.multiple_of(j * CHUNK, 8)
            return idx_v.at[pl.ds(off, CHUNK)]

        def out_slice(j):
            off = pl.multiple_of(base + j * CHUNK, 8)
            return out_hbm.at[pl.ds(off, CHUNK)]

        def gather_start(j, b):
            pltpu.async_copy(table_hbm.at[chunk_idx(j)], rows_v.at[b],
                             g_sem.at[b])

        def gather_wait(j, b):
            pltpu.make_async_copy(table_hbm.at[chunk_idx(j)], rows_v.at[b],
                                  g_sem.at[b]).wait()

        def store_start(j, b):
            pltpu.async_copy(rows_v.at[b], out_slice(j), s_sem.at[b])

        def store_wait(j, b):
            pltpu.make_async_copy(rows_v.at[b], out_slice(j),
                                  s_sem.at[b]).wait()

        for b in range(NBUF):
            gather_start(b, b)

        def body(i, carry):
            for b in range(NBUF):
                j = i * NBUF + b
                gather_wait(j, b)
                store_start(j, b)
                store_wait(j, b)
                gather_start(j + NBUF, b)
            return carry

        lax.fori_loop(0, n_chunks // NBUF - 1, body, 0, unroll=False)

        for b in range(NBUF):
            j = n_chunks - NBUF + b
            gather_wait(j, b)
            store_start(j, b)
            store_wait(j, b)

    return emb_kernel(idx_flat, table)


def kernel(token_ids, embedding_matrix):
    b, s = token_ids.shape
    idx_flat = token_ids.reshape(-1).astype(jnp.int32)
    out = _emb_lookup(idx_flat, embedding_matrix, n_total=b * s)
    return out.reshape(b, s, EMB_DIM)
